# baseline (speedup 1.0000x reference)
import functools

import jax
import jax.numpy as jnp
from jax import lax
from jax.experimental import pallas as pl
from jax.experimental.pallas import tpu as pltpu
from jax.experimental.pallas import tpu_sc as plsc

_LANES = 16
_MAIN = 72


@functools.lru_cache(maxsize=None)
def _make_sc_kernel(n_cls, seq, seqp, dim, num_cores, n_workers, cpw):
    nchunk = dim // _LANES
    ntail = seq - _MAIN
    inv_seq = 1.0 / seq
    mesh = plsc.VectorSubcoreMesh(core_axis_name="c", subcore_axis_name="s")

    @functools.partial(
        pl.kernel,
        mesh=mesh,
        out_type=(
            jax.ShapeDtypeStruct((n_cls, seq, dim), jnp.float32),
            jax.ShapeDtypeStruct((n_cls * dim,), jnp.float32),
        ),
        scratch_types=[
            pltpu.VMEM((_MAIN,), jnp.int32),
            pltpu.VMEM((_MAIN,), jnp.int32),
            pltpu.VMEM((8,), jnp.int32),
            pltpu.VMEM((seq, dim), jnp.float32),
            pltpu.VMEM((seq, dim), jnp.float32),
            pltpu.VMEM((seq, dim), jnp.float32),
            pltpu.VMEM((8, dim), jnp.float32),
            pltpu.VMEM((dim,), jnp.float32),
            pltpu.SemaphoreType.DMA,
            pltpu.SemaphoreType.DMA,
            pltpu.SemaphoreType.DMA,
            pltpu.SemaphoreType.DMA,
            pltpu.SemaphoreType.DMA,
            pltpu.SemaphoreType.DMA,
        ],
    )
    def kfn(idxf_hbm, table_hbm, emb_hbm, mean_hbm,
            idx72_0, idx72_1, idx8, buf0, buf1, buf2, tail, mean1,
            semg0, semg1, semg2, semw0, semw1, semw2):
        wid = lax.axis_index("s") * num_cores + lax.axis_index("c")
        base = wid * cpw
        n = jnp.minimum(cpw, n_cls - base)

        idx72 = (idx72_0, idx72_1)
        buf = (buf0, buf1, buf2)
        semg = (semg0, semg1, semg2)
        semw = (semw0, semw1, semw2)

        def stage_main_idx(c, bi):
            pltpu.sync_copy(idxf_hbm.at[pl.ds((base + c) * seqp, _MAIN)],
                            idx72[bi])

        def fire_main(b, bi):
            pltpu.async_copy(table_hbm.at[idx72[bi]],
                             buf[b].at[pl.ds(0, _MAIN)], semg[b])

        def fire_tail(c, b):
            pltpu.sync_copy(
                idxf_hbm.at[pl.ds((base + c) * seqp + _MAIN, 8)], idx8)
            pltpu.async_copy(table_hbm.at[idx8], tail, semg[b])

        def wait_gathers(b, bi):
            pltpu.make_async_copy(table_hbm.at[idx72[bi]],
                                  buf[b].at[pl.ds(0, _MAIN)], semg[b]).wait()
            pltpu.make_async_copy(table_hbm.at[idx8], tail, semg[b]).wait()

        def wait_write(b):
            pltpu.make_async_copy(buf[b], emb_hbm.at[base], semw[b]).wait()

        def visit(c, k):
            b = k % 3
            b1 = (k + 1) % 3
            bi = k % 2
            bi1 = (k + 1) % 2

            @pl.when(c + 1 < n)
            def _():
                @pl.when(c >= 2)
                def _():
                    wait_write(b1)
                fire_main(b1, bi1)

            wait_gathers(b, bi)
            for r in range(ntail):
                for j in range(nchunk):
                    sl = pl.ds(_LANES * j, _LANES)
                    buf[b][_MAIN + r, sl] = tail[r, sl]

            @pl.when(c + 1 < n)
            def _():
                fire_tail(c + 1, b1)

            @pl.when(c + 2 < n)
            def _():
                stage_main_idx(c + 2, bi)

            accs = []
            for j in range(nchunk):
                accs.append(buf[b][0, pl.ds(_LANES * j, _LANES)])

            def rbody(r, accs):
                return tuple(accs[j] + buf[b][r, pl.ds(_LANES * j, _LANES)]
                             for j in range(nchunk))

            accs = lax.fori_loop(1, seq, rbody, tuple(accs))
            pltpu.async_copy(buf[b], emb_hbm.at[base + c], semw[b])
            for j in range(nchunk):
                mean1[pl.ds(_LANES * j, _LANES)] = accs[j] * inv_seq
            pltpu.sync_copy(mean1, mean_hbm.at[pl.ds((base + c) * dim, dim)])

        stage_main_idx(0, 0)
        fire_main(0, 0)
        fire_tail(0, 0)
        stage_main_idx(1, 1)

        def group_body(g, carry):
            c0 = 6 * g
            for k in range(6):
                @pl.when(c0 + k < n)
                def _(k=k):
                    visit(c0 + k, k)
            return carry

        lax.fori_loop(0, (n + 5) // 6, group_body, 0)
        wait_write(0)
        wait_write(1)
        wait_write(2)

    return kfn


def kernel(tokenized_prompts, token_embedding):
    n_cls, seq = tokenized_prompts.shape
    _, dim = token_embedding.shape
    info = plsc.get_sparse_core_info()
    n_workers = info.num_cores * info.num_subcores
    cpw = -(-n_cls // n_workers)
    seqp = -(-seq // _LANES) * _LANES
    idx = tokenized_prompts.astype(jnp.int32)
    idx_flat = jnp.pad(idx, ((0, 0), (0, seqp - seq))).reshape(-1)
    table = token_embedding.astype(jnp.float32)
    emb, mean_flat = _make_sc_kernel(
        n_cls, seq, seqp, dim, info.num_cores, n_workers, cpw)(idx_flat, table)
    return emb, mean_flat.reshape(n_cls, dim)
